# baseline (speedup 1.0000x reference)
import functools

import jax
import jax.numpy as jnp
from jax.experimental import pallas as pl
from jax.experimental.pallas import tpu as pltpu

_OBS = 8
_ACT = 4
_HID = 64
_TILE = 2048
_STEP = 2 * _TILE


def _ac_kernel(x_ref, w1a_ref, w1b_ref, b1_ref, w2_ref, b2_ref,
               wla_ref, wlb_ref, bl_ref, wva_ref, wvb_ref, bv_ref,
               logits_ref, value_ref):
    xa = x_ref[:_TILE]
    xb = x_ref[_TILE:]
    h1 = jnp.tanh(
        jnp.dot(xa, w1a_ref[...], preferred_element_type=jnp.float32)
        + jnp.dot(xb, w1b_ref[...], preferred_element_type=jnp.float32)
        + b1_ref[...]
    )
    h2 = jnp.tanh(
        jnp.dot(h1, w2_ref[...], preferred_element_type=jnp.float32)
        + b2_ref[...]
    )
    logits_ref[:_TILE] = (
        jnp.dot(h2, wla_ref[...], preferred_element_type=jnp.float32)
        + bl_ref[...]
    )
    logits_ref[_TILE:] = (
        jnp.dot(h2, wlb_ref[...], preferred_element_type=jnp.float32)
        + bl_ref[...]
    )
    value_ref[:_TILE] = (
        jnp.dot(h2, wva_ref[...], preferred_element_type=jnp.float32)
        + bv_ref[...]
    )
    value_ref[_TILE:] = (
        jnp.dot(h2, wvb_ref[...], preferred_element_type=jnp.float32)
        + bv_ref[...]
    )


@functools.partial(jax.jit, static_argnames=("bp",))
def _forward(x, w1a, w1b, b1p, w2p, b2p, wla, wlb, bl, wva, wvb, bv, *, bp):
    grid = (bp // _STEP,)
    logits, value = pl.pallas_call(
        _ac_kernel,
        grid=grid,
        in_specs=[
            pl.BlockSpec((_STEP, _OBS), lambda i: (i, 0)),
            pl.BlockSpec((_OBS, 128), lambda i: (0, 0)),
            pl.BlockSpec((_OBS, 128), lambda i: (0, 0)),
            pl.BlockSpec((1, 128), lambda i: (0, 0)),
            pl.BlockSpec((128, 128), lambda i: (0, 0)),
            pl.BlockSpec((1, 128), lambda i: (0, 0)),
            pl.BlockSpec((128, _ACT), lambda i: (0, 0)),
            pl.BlockSpec((128, _ACT), lambda i: (0, 0)),
            pl.BlockSpec((1, _ACT), lambda i: (0, 0)),
            pl.BlockSpec((128, 1), lambda i: (0, 0)),
            pl.BlockSpec((128, 1), lambda i: (0, 0)),
            pl.BlockSpec((1, 1), lambda i: (0, 0)),
        ],
        out_specs=[
            pl.BlockSpec((_STEP, _ACT), lambda i: (i, 0)),
            pl.BlockSpec((_STEP, 1), lambda i: (i, 0)),
        ],
        out_shape=[
            jax.ShapeDtypeStruct((bp, _ACT), jnp.float32),
            jax.ShapeDtypeStruct((bp, 1), jnp.float32),
        ],
        compiler_params=pltpu.CompilerParams(
            dimension_semantics=("parallel",),
        ),
    )(x, w1a, w1b, b1p, w2p, b2p, wla, wlb, bl, wva, wvb, bv)
    return logits, value


def kernel(x, w1, b1, w2, b2, wh, bh):
    B = x.shape[0]
    bp = -(-B // _STEP) * _STEP
    if bp != B:
        x = jnp.pad(x, ((0, bp - B), (0, 0)))

    w1c = w1[:, :_HID]
    w1a = jnp.zeros((_OBS, 128), jnp.float32).at[:, :_HID].set(w1c)
    w1b = jnp.zeros((_OBS, 128), jnp.float32).at[:, _HID:].set(w1c)
    b1c = b1[:, :_HID]
    b1p = jnp.concatenate([b1c, b1c], axis=1)
    w2c = w2[:_HID, :_HID]
    w2p = (jnp.zeros((128, 128), jnp.float32)
           .at[:_HID, :_HID].set(w2c)
           .at[_HID:, _HID:].set(w2c))
    b2c = b2[:, :_HID]
    b2p = jnp.concatenate([b2c, b2c], axis=1)
    wa = wh[:_HID, :_ACT]
    wla = jnp.zeros((128, _ACT), jnp.float32).at[:_HID].set(wa)
    wlb = jnp.zeros((128, _ACT), jnp.float32).at[_HID:].set(wa)
    bl = bh[:, :_ACT]
    wc = wh[:_HID, _ACT:_ACT + 1]
    wva = jnp.zeros((128, 1), jnp.float32).at[:_HID].set(wc)
    wvb = jnp.zeros((128, 1), jnp.float32).at[_HID:].set(wc)
    bv = bh[:, _ACT:_ACT + 1]

    logits, value = _forward(
        x, w1a, w1b, b1p, w2p, b2p, wla, wlb, bl, wva, wvb, bv, bp=bp)
    if bp != B:
        logits = logits[:B]
        value = value[:B]
    return logits, value


# speedup vs baseline: 1.2192x; 1.2192x over previous
import functools

import jax
import jax.numpy as jnp
from jax.experimental import pallas as pl
from jax.experimental.pallas import tpu as pltpu

_OBS = 8
_ACT = 4
_HID = 64
_TILE = 2048


def _ac_kernel(xa_ref, xb_ref, w1a_ref, w1b_ref, b1_ref, w2_ref, b2_ref,
               wla_ref, wlb_ref, bl_ref, wva_ref, wvb_ref, bv_ref,
               logits_ref, value_ref):
    h1 = jnp.tanh(
        jnp.dot(xa_ref[...], w1a_ref[...], preferred_element_type=jnp.float32)
        + jnp.dot(xb_ref[...], w1b_ref[...], preferred_element_type=jnp.float32)
        + b1_ref[...]
    )
    h2 = jnp.tanh(
        jnp.dot(h1, w2_ref[...], preferred_element_type=jnp.float32)
        + b2_ref[...]
    )
    logits_ref[0] = (
        jnp.dot(h2, wla_ref[...], preferred_element_type=jnp.float32)
        + bl_ref[...]
    )
    logits_ref[1] = (
        jnp.dot(h2, wlb_ref[...], preferred_element_type=jnp.float32)
        + bl_ref[...]
    )
    value_ref[0] = (
        jnp.dot(h2, wva_ref[...], preferred_element_type=jnp.float32)
        + bv_ref[...]
    )
    value_ref[1] = (
        jnp.dot(h2, wvb_ref[...], preferred_element_type=jnp.float32)
        + bv_ref[...]
    )


@functools.partial(jax.jit, static_argnames=("half",))
def _forward(x, w1a, w1b, b1p, w2p, b2p, wla, wlb, bl, wva, wvb, bv, *, half):
    n_half = half // _TILE
    grid = (n_half,)
    logits3, value3 = pl.pallas_call(
        _ac_kernel,
        grid=grid,
        in_specs=[
            pl.BlockSpec((_TILE, _OBS), lambda i: (i, 0)),
            pl.BlockSpec((_TILE, _OBS), lambda i: (i + n_half, 0)),
            pl.BlockSpec((_OBS, 128), lambda i: (0, 0)),
            pl.BlockSpec((_OBS, 128), lambda i: (0, 0)),
            pl.BlockSpec((1, 128), lambda i: (0, 0)),
            pl.BlockSpec((128, 128), lambda i: (0, 0)),
            pl.BlockSpec((1, 128), lambda i: (0, 0)),
            pl.BlockSpec((128, _ACT), lambda i: (0, 0)),
            pl.BlockSpec((128, _ACT), lambda i: (0, 0)),
            pl.BlockSpec((1, _ACT), lambda i: (0, 0)),
            pl.BlockSpec((128, 1), lambda i: (0, 0)),
            pl.BlockSpec((128, 1), lambda i: (0, 0)),
            pl.BlockSpec((1, 1), lambda i: (0, 0)),
        ],
        out_specs=[
            pl.BlockSpec((2, _TILE, _ACT), lambda i: (0, i, 0)),
            pl.BlockSpec((2, _TILE, 1), lambda i: (0, i, 0)),
        ],
        out_shape=[
            jax.ShapeDtypeStruct((2, half, _ACT), jnp.float32),
            jax.ShapeDtypeStruct((2, half, 1), jnp.float32),
        ],
        compiler_params=pltpu.CompilerParams(
            dimension_semantics=("parallel",),
        ),
    )(x, x, w1a, w1b, b1p, w2p, b2p, wla, wlb, bl, wva, wvb, bv)
    return logits3, value3


def kernel(x, w1, b1, w2, b2, wh, bh):
    B = x.shape[0]
    half = -(-B // (2 * _TILE)) * _TILE
    if 2 * half != B:
        x = jnp.pad(x, ((0, 2 * half - B), (0, 0)))

    w1c = w1[:, :_HID]
    w1a = jnp.zeros((_OBS, 128), jnp.float32).at[:, :_HID].set(w1c)
    w1b = jnp.zeros((_OBS, 128), jnp.float32).at[:, _HID:].set(w1c)
    b1c = b1[:, :_HID]
    b1p = jnp.concatenate([b1c, b1c], axis=1)
    w2c = w2[:_HID, :_HID]
    w2p = (jnp.zeros((128, 128), jnp.float32)
           .at[:_HID, :_HID].set(w2c)
           .at[_HID:, _HID:].set(w2c))
    b2c = b2[:, :_HID]
    b2p = jnp.concatenate([b2c, b2c], axis=1)
    wa = wh[:_HID, :_ACT]
    wla = jnp.zeros((128, _ACT), jnp.float32).at[:_HID].set(wa)
    wlb = jnp.zeros((128, _ACT), jnp.float32).at[_HID:].set(wa)
    bl = bh[:, :_ACT]
    wc = wh[:_HID, _ACT:_ACT + 1]
    wva = jnp.zeros((128, 1), jnp.float32).at[:_HID].set(wc)
    wvb = jnp.zeros((128, 1), jnp.float32).at[_HID:].set(wc)
    bv = bh[:, _ACT:_ACT + 1]

    logits3, value3 = _forward(
        x, w1a, w1b, b1p, w2p, b2p, wla, wlb, bl, wva, wvb, bv, half=half)
    logits = logits3.reshape(2 * half, _ACT)[:B]
    value = value3.reshape(2 * half, 1)[:B]
    return logits, value


# speedup vs baseline: 1.6269x; 1.3344x over previous
import functools

import jax
import jax.numpy as jnp
from jax.experimental import pallas as pl
from jax.experimental.pallas import tpu as pltpu

_OBS = 8
_ACT = 4
_HID = 64
_TILE = 4096


def _ac_kernel(x_ref, w1a_ref, w1b_ref, b1_ref, w2_ref, b2_ref,
               wla_ref, wlb_ref, bl_ref, wva_ref, wvb_ref, bv_ref,
               logits_ref, value_ref):
    xa = x_ref[0]
    xb = x_ref[1]
    h1 = jnp.tanh(
        jnp.dot(xa, w1a_ref[...], preferred_element_type=jnp.float32)
        + jnp.dot(xb, w1b_ref[...], preferred_element_type=jnp.float32)
        + b1_ref[...]
    )
    h2 = jnp.tanh(
        jnp.dot(h1, w2_ref[...], preferred_element_type=jnp.float32)
        + b2_ref[...]
    )
    logits_ref[0] = (
        jnp.dot(h2, wla_ref[...], preferred_element_type=jnp.float32)
        + bl_ref[...]
    )
    logits_ref[1] = (
        jnp.dot(h2, wlb_ref[...], preferred_element_type=jnp.float32)
        + bl_ref[...]
    )
    value_ref[0] = (
        jnp.dot(h2, wva_ref[...], preferred_element_type=jnp.float32)
        + bv_ref[...]
    )
    value_ref[1] = (
        jnp.dot(h2, wvb_ref[...], preferred_element_type=jnp.float32)
        + bv_ref[...]
    )


@functools.partial(jax.jit, static_argnames=("half",))
def _forward(x3, w1a, w1b, b1p, w2p, b2p, wla, wlb, bl, wva, wvb, bv, *, half):
    grid = (half // _TILE,)
    logits3, value3 = pl.pallas_call(
        _ac_kernel,
        grid=grid,
        in_specs=[
            pl.BlockSpec((2, _TILE, _OBS), lambda i: (0, i, 0)),
            pl.BlockSpec((_OBS, 128), lambda i: (0, 0)),
            pl.BlockSpec((_OBS, 128), lambda i: (0, 0)),
            pl.BlockSpec((1, 128), lambda i: (0, 0)),
            pl.BlockSpec((128, 128), lambda i: (0, 0)),
            pl.BlockSpec((1, 128), lambda i: (0, 0)),
            pl.BlockSpec((128, _ACT), lambda i: (0, 0)),
            pl.BlockSpec((128, _ACT), lambda i: (0, 0)),
            pl.BlockSpec((1, _ACT), lambda i: (0, 0)),
            pl.BlockSpec((128, 1), lambda i: (0, 0)),
            pl.BlockSpec((128, 1), lambda i: (0, 0)),
            pl.BlockSpec((1, 1), lambda i: (0, 0)),
        ],
        out_specs=[
            pl.BlockSpec((2, _TILE, _ACT), lambda i: (0, i, 0)),
            pl.BlockSpec((2, _TILE, 1), lambda i: (0, i, 0)),
        ],
        out_shape=[
            jax.ShapeDtypeStruct((2, half, _ACT), jnp.float32),
            jax.ShapeDtypeStruct((2, half, 1), jnp.float32),
        ],
        compiler_params=pltpu.CompilerParams(
            dimension_semantics=("parallel",),
        ),
    )(x3, w1a, w1b, b1p, w2p, b2p, wla, wlb, bl, wva, wvb, bv)
    return logits3, value3


def kernel(x, w1, b1, w2, b2, wh, bh):
    B = x.shape[0]
    half = -(-B // (2 * _TILE)) * _TILE
    if 2 * half != B:
        x = jnp.pad(x, ((0, 2 * half - B), (0, 0)))
    x3 = x.reshape(2, half, _OBS)

    w1c = w1[:, :_HID]
    w1a = jnp.zeros((_OBS, 128), jnp.float32).at[:, :_HID].set(w1c)
    w1b = jnp.zeros((_OBS, 128), jnp.float32).at[:, _HID:].set(w1c)
    b1c = b1[:, :_HID]
    b1p = jnp.concatenate([b1c, b1c], axis=1)
    w2c = w2[:_HID, :_HID]
    w2p = (jnp.zeros((128, 128), jnp.float32)
           .at[:_HID, :_HID].set(w2c)
           .at[_HID:, _HID:].set(w2c))
    b2c = b2[:, :_HID]
    b2p = jnp.concatenate([b2c, b2c], axis=1)
    wa = wh[:_HID, :_ACT]
    wla = jnp.zeros((128, _ACT), jnp.float32).at[:_HID].set(wa)
    wlb = jnp.zeros((128, _ACT), jnp.float32).at[_HID:].set(wa)
    bl = bh[:, :_ACT]
    wc = wh[:_HID, _ACT:_ACT + 1]
    wva = jnp.zeros((128, 1), jnp.float32).at[:_HID].set(wc)
    wvb = jnp.zeros((128, 1), jnp.float32).at[_HID:].set(wc)
    bv = bh[:, _ACT:_ACT + 1]

    logits3, value3 = _forward(
        x3, w1a, w1b, b1p, w2p, b2p, wla, wlb, bl, wva, wvb, bv, half=half)
    logits = logits3.reshape(2 * half, _ACT)[:B]
    value = value3.reshape(2 * half, 1)[:B]
    return logits, value
